# baseline (speedup 1.0000x reference)
import functools

import jax
import jax.numpy as jnp
from jax import lax
from jax.experimental import pallas as pl
from jax.experimental.pallas import tpu as pltpu
from jax.experimental.pallas import tpu_sc as plsc

B = 4
N = 2048
K = 1024
TILE = 256


def _rank_kernel(conf_row_ref, conf_col_ref, lg_ref, preds_ref, idx_ref):
    conf_row = conf_row_ref[0]
    iota_row = lax.broadcasted_iota(jnp.int32, (1, N), 1)

    ones_row = jnp.ones((1, TILE), jnp.bfloat16)
    rankf = jnp.zeros((1, N), jnp.float32)
    for t in range(N // TILE):
        cj = conf_col_ref[0, t * TILE:(t + 1) * TILE, 0:1]
        jj = lax.broadcasted_iota(jnp.int32, (TILE, 1), 0) + t * TILE
        gef = (cj >= conf_row).astype(jnp.bfloat16)
        gtf = (cj > conf_row).astype(jnp.bfloat16)
        maskh = jnp.where(jj < iota_row, gef, gtf)
        rankf = rankf + lax.dot_general(
            ones_row, maskh, (((1,), (0,)), ((), ())),
            preferred_element_type=jnp.float32)

    lg = lg_ref[0]
    lg_hi = lg.astype(jnp.bfloat16)
    r1 = lg - lg_hi.astype(jnp.float32)
    lg_mid = r1.astype(jnp.bfloat16)
    lg_lo = (r1 - lg_mid.astype(jnp.float32)).astype(jnp.bfloat16)
    ii = lax.broadcasted_iota(jnp.int32, (N, 1), 0)
    ia = (ii // 256).astype(jnp.bfloat16)
    ic = (ii % 256).astype(jnp.bfloat16)
    vals = jnp.concatenate(
        [lg_hi, lg_mid, lg_lo, ia, ic], axis=1)

    for t in range(N // TILE):
        rr = (lax.broadcasted_iota(jnp.int32, (TILE, 1), 0)
              + t * TILE).astype(jnp.float32)
        oh = (rankf == rr).astype(jnp.bfloat16)
        out = lax.dot_general(
            oh, vals, (((1,), (0,)), ((), ())),
            preferred_element_type=jnp.float32)
        preds_ref[0, t * TILE:(t + 1) * TILE, :] = (
            out[:, 0:2] + out[:, 2:4] + out[:, 4:6])
        idx_ref[0, t * TILE:(t + 1) * TILE, :] = (
            256 * out[:, 6:7] + out[:, 7:8]).astype(jnp.int32)


_NC = 2
_NS = 16
_NW = _NC * _NS
_FPW = K // 8
_CH = 16
_NCH = _FPW // _CH


def _sc_body(x_hbm, idx_hbm, sf_hbm, fidx_v, buf_v,
             gsem_a, gsem_b, osem_a, osem_b):
    cid = lax.axis_index("c")
    sid = lax.axis_index("s")
    wid = sid * _NC + cid

    bb = wid // 8
    x_b = x_hbm.at[bb]
    sf_b = sf_hbm.at[bb]
    obase = (wid % 8) * _FPW
    pltpu.sync_copy(idx_hbm.at[bb].at[pl.ds(obase, _FPW)], fidx_v)
    gsems = (gsem_a, gsem_b)
    osems = (osem_a, osem_b)
    gd = [None, None]
    od = [None, None]
    for ch in range(_NCH):
        s = ch % 2
        if od[s] is not None:
            od[s].wait()
        gd[s] = pltpu.async_copy(
            x_b.at[fidx_v.at[pl.ds(ch * _CH, _CH)]],
            buf_v.at[s], gsems[s])
        if ch > 0:
            p = 1 - s
            gd[p].wait()
            od[p] = pltpu.async_copy(
                buf_v.at[p],
                sf_b.at[pl.ds(obase + (ch - 1) * _CH, _CH)], osems[p])
    s = (_NCH - 1) % 2
    gd[s].wait()
    od[s] = pltpu.async_copy(
        buf_v.at[s], sf_b.at[pl.ds(obase + (_NCH - 1) * _CH, _CH)],
        osems[s])
    od[0].wait()
    od[1].wait()


def kernel(x_feat, logits_feat):
    probs = jax.nn.softmax(logits_feat, axis=-1)
    conf = jnp.max(probs, axis=-1)
    conf_row = conf.reshape(B, 1, N)
    conf_col = conf.reshape(B, N, 1)

    preds, idx = pl.pallas_call(
        _rank_kernel,
        grid=(B,),
        in_specs=[
            pl.BlockSpec((1, 1, N), lambda b: (b, 0, 0)),
            pl.BlockSpec((1, N, 1), lambda b: (b, 0, 0)),
            pl.BlockSpec((1, N, 2), lambda b: (b, 0, 0)),
        ],
        out_specs=[
            pl.BlockSpec((1, N, 2), lambda b: (b, 0, 0)),
            pl.BlockSpec((1, N, 1), lambda b: (b, 0, 0)),
        ],
        out_shape=[
            jax.ShapeDtypeStruct((B, N, 2), jnp.float32),
            jax.ShapeDtypeStruct((B, N, 1), jnp.int32),
        ],
    )(conf_row, conf_col, logits_feat)

    D = x_feat.shape[-1]
    gather_call = functools.partial(
        pl.kernel,
        mesh=plsc.VectorSubcoreMesh(core_axis_name="c", subcore_axis_name="s"),
        out_type=[
            jax.ShapeDtypeStruct((B, K, D), jnp.float32),
        ],
        scratch_types=[
            pltpu.VMEM((_FPW,), jnp.int32),
            pltpu.VMEM((2, _CH, D), jnp.float32),
            pltpu.SemaphoreType.DMA,
            pltpu.SemaphoreType.DMA,
            pltpu.SemaphoreType.DMA,
            pltpu.SemaphoreType.DMA,
        ],
    )(_sc_body)
    (sf,) = gather_call(x_feat, idx.reshape(B, N))
    return sf, preds[:, :K, :], preds[:, K:, :]


# speedup vs baseline: 1.0408x; 1.0408x over previous
import functools

import jax
import jax.numpy as jnp
from jax import lax
from jax.experimental import pallas as pl
from jax.experimental.pallas import tpu as pltpu
from jax.experimental.pallas import tpu_sc as plsc

B = 4
N = 2048
K = 1024
TILE = 256


def _rank_kernel(conf_row_ref, conf_col_ref, lg_ref, preds_ref, idx_ref):
    conf_row = conf_row_ref[0]
    iota_row = lax.broadcasted_iota(jnp.int32, (1, N), 1)

    ones_row = jnp.ones((1, TILE), jnp.bfloat16)
    rankf = jnp.zeros((1, N), jnp.float32)
    for t in range(N // TILE):
        cj = conf_col_ref[0, t * TILE:(t + 1) * TILE, 0:1]
        jj = lax.broadcasted_iota(jnp.int32, (TILE, 1), 0) + t * TILE
        gt = cj > conf_row
        eqb = (cj == conf_row) & (jj < iota_row)
        maskh = (gt | eqb).astype(jnp.bfloat16)
        rankf = rankf + lax.dot_general(
            ones_row, maskh, (((1,), (0,)), ((), ())),
            preferred_element_type=jnp.float32)

    lg = lg_ref[0]
    lg_hi = lg.astype(jnp.bfloat16)
    r1 = lg - lg_hi.astype(jnp.float32)
    lg_mid = r1.astype(jnp.bfloat16)
    lg_lo = (r1 - lg_mid.astype(jnp.float32)).astype(jnp.bfloat16)
    ii = lax.broadcasted_iota(jnp.int32, (N, 1), 0)
    ia = (ii // 256).astype(jnp.bfloat16)
    ic = (ii % 256).astype(jnp.bfloat16)
    vals = jnp.concatenate(
        [lg_hi, lg_mid, lg_lo, ia, ic], axis=1)

    for t in range(N // TILE):
        rr = (lax.broadcasted_iota(jnp.int32, (TILE, 1), 0)
              + t * TILE).astype(jnp.float32)
        oh = (rankf == rr).astype(jnp.bfloat16)
        out = lax.dot_general(
            oh, vals, (((1,), (0,)), ((), ())),
            preferred_element_type=jnp.float32)
        preds_ref[0, t * TILE:(t + 1) * TILE, :] = (
            out[:, 0:2] + out[:, 2:4] + out[:, 4:6])
        idx_ref[0, t * TILE:(t + 1) * TILE, :] = (
            256 * out[:, 6:7] + out[:, 7:8]).astype(jnp.int32)


_NC = 2
_NS = 16
_NW = _NC * _NS
_FPW = K // 8
_CH = 16
_NCH = _FPW // _CH


def _sc_body(x_hbm, idx_hbm, sf_hbm, fidx_v, buf_v,
             gsem_a, gsem_b, osem_a, osem_b):
    cid = lax.axis_index("c")
    sid = lax.axis_index("s")
    wid = sid * _NC + cid

    bb = wid // 8
    x_b = x_hbm.at[bb]
    sf_b = sf_hbm.at[bb]
    obase = (wid % 8) * _FPW
    pltpu.sync_copy(idx_hbm.at[bb].at[pl.ds(obase, _FPW)], fidx_v)
    gsems = (gsem_a, gsem_b)
    osems = (osem_a, osem_b)
    gd = [None, None]
    od = [None, None]
    for ch in range(_NCH):
        s = ch % 2
        if od[s] is not None:
            od[s].wait()
        gd[s] = pltpu.async_copy(
            x_b.at[fidx_v.at[pl.ds(ch * _CH, _CH)]],
            buf_v.at[s], gsems[s])
        if ch > 0:
            p = 1 - s
            gd[p].wait()
            od[p] = pltpu.async_copy(
                buf_v.at[p],
                sf_b.at[pl.ds(obase + (ch - 1) * _CH, _CH)], osems[p])
    s = (_NCH - 1) % 2
    gd[s].wait()
    od[s] = pltpu.async_copy(
        buf_v.at[s], sf_b.at[pl.ds(obase + (_NCH - 1) * _CH, _CH)],
        osems[s])
    od[0].wait()
    od[1].wait()


def kernel(x_feat, logits_feat):
    probs = jax.nn.softmax(logits_feat, axis=-1)
    conf = jnp.max(probs, axis=-1)
    conf_row = conf.reshape(B, 1, N)
    conf_col = conf.reshape(B, N, 1)

    preds, idx = pl.pallas_call(
        _rank_kernel,
        grid=(B,),
        in_specs=[
            pl.BlockSpec((1, 1, N), lambda b: (b, 0, 0)),
            pl.BlockSpec((1, N, 1), lambda b: (b, 0, 0)),
            pl.BlockSpec((1, N, 2), lambda b: (b, 0, 0)),
        ],
        out_specs=[
            pl.BlockSpec((1, N, 2), lambda b: (b, 0, 0)),
            pl.BlockSpec((1, N, 1), lambda b: (b, 0, 0)),
        ],
        out_shape=[
            jax.ShapeDtypeStruct((B, N, 2), jnp.float32),
            jax.ShapeDtypeStruct((B, N, 1), jnp.int32),
        ],
    )(conf_row, conf_col, logits_feat)

    D = x_feat.shape[-1]
    gather_call = functools.partial(
        pl.kernel,
        mesh=plsc.VectorSubcoreMesh(core_axis_name="c", subcore_axis_name="s"),
        out_type=[
            jax.ShapeDtypeStruct((B, K, D), jnp.float32),
        ],
        scratch_types=[
            pltpu.VMEM((_FPW,), jnp.int32),
            pltpu.VMEM((2, _CH, D), jnp.float32),
            pltpu.SemaphoreType.DMA,
            pltpu.SemaphoreType.DMA,
            pltpu.SemaphoreType.DMA,
            pltpu.SemaphoreType.DMA,
        ],
    )(_sc_body)
    (sf,) = gather_call(x_feat, idx.reshape(B, N))
    return sf, preds[:, :K, :], preds[:, K:, :]
